# baseline (speedup 1.0000x reference)
import functools

import jax
import jax.numpy as jnp
from jax import lax
from jax.experimental import pallas as pl
from jax.experimental.pallas import tpu as pltpu, tpu_sc as plsc

_N_NODES = 1_000_000
_MEM_DIM = 128
_BATCH = 16384

_NC = 2
_NS = 16
_L = 16
_SLICE = 62504
_TBL = _NS * _SLICE
_ROWS_PER_TILE = _BATCH // (_NC * _NS)
_CHUNK = 128
_VREGS = _BATCH // _L

_SPEC = 64
_NSEG = _ROWS_PER_TILE // _SPEC
_RING = 4
_UNROLL = 8
_SEG_ITERS = _VREGS // (_NSEG * _UNROLL)


_mesh = plsc.VectorSubcoreMesh(core_axis_name="c", subcore_axis_name="s")


@functools.partial(
    pl.kernel,
    mesh=_mesh,
    out_type=[
        jax.ShapeDtypeStruct((_BATCH, _MEM_DIM), jnp.float32),
        jax.ShapeDtypeStruct((_NC * _TBL,), jnp.int32),
    ],
    scratch_types=[
        pltpu.VMEM((_BATCH,), jnp.int32),
        pltpu.VMEM((_SLICE,), jnp.int32),
        pltpu.VMEM((_ROWS_PER_TILE,), jnp.int32),
        pltpu.VMEM((_ROWS_PER_TILE,), jnp.int32),
        pltpu.VMEM((_RING, _SPEC, _MEM_DIM), jnp.float32),
        pltpu.SemaphoreType.DMA,
        pltpu.SemaphoreType.DMA,
        pltpu.SemaphoreType.DMA,
        pltpu.SemaphoreType.DMA,
    ],
    compiler_params=pltpu.CompilerParams(needs_layout_passes=False),
)
def _sc_scatter_gather(ids_hbm, values_hbm, out_hbm, table_hbm,
                       ids_v, tbl_v, tidx_v, widx_v, rows_v,
                       sem_r, sem_o, sem_t, sem_f):
    c = lax.axis_index("c")
    s = lax.axis_index("s")
    w = c * _NS + s
    base = w * _ROWS_PER_TILE

    pltpu.sync_copy(ids_hbm, ids_v)

    lane = lax.iota(jnp.int32, _L)
    lo = s * _SLICE

    def spec_read(k):
        return pltpu.async_copy(
            values_hbm.at[pl.ds(base + k * _SPEC, _SPEC)],
            rows_v.at[k % _RING], sem_r)

    rdesc = {k: spec_read(k) for k in range(_RING)}
    wdesc = {}

    def build(k, carry):
        for u in range(_UNROLL):
            v = k * _UNROLL + u
            loc = ids_v[pl.ds(v * _L, _L)] - lo
            mine = loc.astype(jnp.uint32) < jnp.uint32(_SLICE)
            plsc.store_scatter(tbl_v, [loc], v * _L + lane, mask=mine)
        return carry

    for seg in range(_NSEG):
        rdesc[seg].wait()
        wdesc[seg] = pltpu.async_copy(
            rows_v.at[seg % _RING],
            out_hbm.at[pl.ds(base + seg * _SPEC, _SPEC)], sem_o)
        off0 = seg * _SEG_ITERS
        lax.fori_loop(off0, off0 + _SEG_ITERS, build, 0)
        wdesc[seg].wait()
        if seg + _RING < _NSEG:
            rdesc[seg + _RING] = spec_read(seg + _RING)

    tbl_base = pl.multiple_of(c * _TBL + lo, 8)
    pltpu.sync_copy(tbl_v, table_hbm.at[pl.ds(tbl_base, _SLICE)])
    plsc.subcore_barrier()

    for t in range(_ROWS_PER_TILE // _L):
        tidx_v[pl.ds(t * _L, _L)] = (
            ids_v[pl.ds(base + t * _L, _L)] + c * _TBL)
    tdesc = [pltpu.async_copy(
        table_hbm.at[tidx_v.at[pl.ds(g * _CHUNK, _CHUNK)]],
        widx_v.at[pl.ds(g * _CHUNK, _CHUNK)], sem_t)
        for g in range(_ROWS_PER_TILE // _CHUNK)]
    for d in tdesc:
        d.wait()

    for t in range(_ROWS_PER_TILE // _L):
        wv = widx_v[pl.ds(t * _L, _L)]
        iv = base + t * _L + lane
        cnt = jnp.sum((wv != iv).astype(jnp.int32))

        @pl.when(cnt > 0)
        def _fix(t=t):
            pltpu.async_copy(
                values_hbm.at[widx_v.at[pl.ds(t * _L, _L)]],
                rows_v.at[0, pl.ds(0, _L)], sem_f).wait()
            pltpu.sync_copy(rows_v.at[0, pl.ds(0, _L)],
                            out_hbm.at[pl.ds(base + t * _L, _L)])


def kernel(memory, node_ids, values):
    del memory
    out, _ = _sc_scatter_gather(node_ids, values)
    return out


# speedup vs baseline: 1.2106x; 1.2106x over previous
import functools

import jax
import jax.numpy as jnp
from jax import lax
from jax.experimental import pallas as pl
from jax.experimental.pallas import tpu as pltpu, tpu_sc as plsc

_N_NODES = 1_000_000
_MEM_DIM = 128
_BATCH = 16384

_NC = 2
_NS = 16
_L = 16
_SLICE = 62504
_TBL = _NS * _SLICE
_ROWS_PER_TILE = _BATCH // (_NC * _NS)
_CHUNK = 128
_VREGS = _BATCH // _L


_mesh = plsc.VectorSubcoreMesh(core_axis_name="c", subcore_axis_name="s")


@functools.partial(
    pl.kernel,
    mesh=_mesh,
    out_type=[
        jax.ShapeDtypeStruct((_BATCH, _MEM_DIM), jnp.float32),
        jax.ShapeDtypeStruct((_NC * _TBL,), jnp.int32),
    ],
    scratch_types=[
        pltpu.VMEM((_BATCH,), jnp.int32),
        pltpu.VMEM((_SLICE,), jnp.int32),
        pltpu.VMEM((_ROWS_PER_TILE,), jnp.int32),
        pltpu.VMEM((_ROWS_PER_TILE,), jnp.int32),
        pltpu.VMEM((3, _CHUNK, _MEM_DIM), jnp.float32),
        pltpu.SemaphoreType.DMA,
        pltpu.SemaphoreType.DMA,
        pltpu.SemaphoreType.DMA,
    ],
    compiler_params=pltpu.CompilerParams(needs_layout_passes=False),
)
def _sc_scatter_gather(ids_hbm, values_hbm, out_hbm, table_hbm,
                       ids_v, tbl_v, tidx_v, widx_v, rows_v,
                       sem_t, sem_r, sem_o):
    c = lax.axis_index("c")
    s = lax.axis_index("s")
    w = c * _NS + s

    half = _BATCH // 2
    hdesc = [pltpu.async_copy(ids_hbm.at[pl.ds(h * half, half)],
                              ids_v.at[pl.ds(h * half, half)], sem_t)
             for h in range(2)]

    lane = lax.iota(jnp.int32, _L)
    lo = s * _SLICE

    _UNROLL = 8

    def build(k, carry):
        for u in range(_UNROLL):
            v = k * _UNROLL + u
            loc = ids_v[pl.ds(v * _L, _L)] - lo
            mine = loc.astype(jnp.uint32) < jnp.uint32(_SLICE)
            plsc.store_scatter(tbl_v, [loc], v * _L + lane, mask=mine)
        return carry

    n_half = _VREGS // (2 * _UNROLL)
    hdesc[0].wait()
    lax.fori_loop(0, n_half, build, 0)
    hdesc[1].wait()
    lax.fori_loop(n_half, 2 * n_half, build, 0)

    tbl_base = pl.multiple_of(c * _TBL + lo, 8)
    cdesc = pltpu.async_copy(tbl_v, table_hbm.at[pl.ds(tbl_base, _SLICE)],
                             sem_o)

    base = w * _ROWS_PER_TILE
    n_chunks = _ROWS_PER_TILE // _CHUNK

    for t in range(_ROWS_PER_TILE // _L):
        tidx_v[pl.ds(t * _L, _L)] = (
            ids_v[pl.ds(base + t * _L, _L)] + c * _TBL)
    cdesc.wait()
    plsc.subcore_barrier()
    tdesc = [pltpu.async_copy(table_hbm.at[tidx_v.at[pl.ds(g * _CHUNK, _CHUNK)]],
                              widx_v.at[pl.ds(g * _CHUNK, _CHUNK)], sem_t)
             for g in range(n_chunks)]
    for d in tdesc:
        d.wait()

    def fire_rows(g):
        return pltpu.async_copy(
            values_hbm.at[widx_v.at[pl.ds(g * _CHUNK, _CHUNK)]],
            rows_v.at[g % 3], sem_r)

    rdesc = {g: fire_rows(g) for g in range(min(3, n_chunks))}
    odesc = {}
    for g in range(n_chunks):
        rdesc[g].wait()
        odesc[g] = pltpu.async_copy(
            rows_v.at[g % 3], out_hbm.at[pl.ds(base + g * _CHUNK, _CHUNK)],
            sem_o)
        if g + 3 < n_chunks:
            odesc[g].wait()
            rdesc[g + 3] = fire_rows(g + 3)
    for g in range(max(0, n_chunks - 3), n_chunks):
        odesc[g].wait()


def kernel(memory, node_ids, values):
    del memory
    out, _ = _sc_scatter_gather(node_ids, values)
    return out


# speedup vs baseline: 1.2243x; 1.0113x over previous
import functools

import jax
import jax.numpy as jnp
from jax import lax
from jax.experimental import pallas as pl
from jax.experimental.pallas import tpu as pltpu, tpu_sc as plsc

_N_NODES = 1_000_000
_MEM_DIM = 128
_BATCH = 16384

_NC = 2
_NS = 16
_L = 16
_SLICE = 62504
_TBL = _NS * _SLICE
_ROWS_PER_TILE = _BATCH // (_NC * _NS)
_CHUNK = 128
_VREGS = _BATCH // _L


_mesh = plsc.VectorSubcoreMesh(core_axis_name="c", subcore_axis_name="s")


@functools.partial(
    pl.kernel,
    mesh=_mesh,
    out_type=[
        jax.ShapeDtypeStruct((_BATCH, _MEM_DIM), jnp.float32),
        jax.ShapeDtypeStruct((_NC * _TBL,), jnp.int32),
    ],
    scratch_types=[
        pltpu.VMEM((_BATCH,), jnp.int32),
        pltpu.VMEM((_SLICE + 8,), jnp.int32),
        pltpu.VMEM((_ROWS_PER_TILE,), jnp.int32),
        pltpu.VMEM((_ROWS_PER_TILE,), jnp.int32),
        pltpu.VMEM((3, _CHUNK, _MEM_DIM), jnp.float32),
        pltpu.SemaphoreType.DMA,
        pltpu.SemaphoreType.DMA,
        pltpu.SemaphoreType.DMA,
    ],
    compiler_params=pltpu.CompilerParams(needs_layout_passes=False),
)
def _sc_scatter_gather(ids_hbm, values_hbm, out_hbm, table_hbm,
                       ids_v, tbl_v, tidx_v, widx_v, rows_v,
                       sem_t, sem_r, sem_o):
    c = lax.axis_index("c")
    s = lax.axis_index("s")
    w = c * _NS + s

    half = _BATCH // 2
    hdesc = [pltpu.async_copy(ids_hbm.at[pl.ds(h * half, half)],
                              ids_v.at[pl.ds(h * half, half)], sem_t)
             for h in range(2)]

    lane = lax.iota(jnp.int32, _L)
    lo = s * _SLICE

    _UNROLL = 8

    def build(k, carry):
        for u in range(_UNROLL):
            v = k * _UNROLL + u
            loc = ids_v[pl.ds(v * _L, _L)] - lo
            locu = plsc.bitcast(loc, jnp.uint32)
            idx = plsc.bitcast(jnp.minimum(locu, jnp.uint32(_SLICE)),
                               jnp.int32)
            plsc.store_scatter(tbl_v, [idx], v * _L + lane)
        return carry

    n_half = _VREGS // (2 * _UNROLL)
    hdesc[0].wait()
    lax.fori_loop(0, n_half, build, 0)
    hdesc[1].wait()
    lax.fori_loop(n_half, 2 * n_half, build, 0)

    tbl_base = pl.multiple_of(c * _TBL + lo, 8)
    cdesc = pltpu.async_copy(tbl_v.at[pl.ds(0, _SLICE)],
                             table_hbm.at[pl.ds(tbl_base, _SLICE)], sem_o)

    base = w * _ROWS_PER_TILE
    n_chunks = _ROWS_PER_TILE // _CHUNK

    for t in range(_ROWS_PER_TILE // _L):
        tidx_v[pl.ds(t * _L, _L)] = (
            ids_v[pl.ds(base + t * _L, _L)] + c * _TBL)
    cdesc.wait()
    plsc.subcore_barrier()
    tdesc = [pltpu.async_copy(table_hbm.at[tidx_v.at[pl.ds(g * _CHUNK, _CHUNK)]],
                              widx_v.at[pl.ds(g * _CHUNK, _CHUNK)], sem_t)
             for g in range(n_chunks)]
    for d in tdesc:
        d.wait()

    def fire_rows(g):
        return pltpu.async_copy(
            values_hbm.at[widx_v.at[pl.ds(g * _CHUNK, _CHUNK)]],
            rows_v.at[g % 3], sem_r)

    rdesc = {g: fire_rows(g) for g in range(min(3, n_chunks))}
    odesc = {}
    for g in range(n_chunks):
        rdesc[g].wait()
        odesc[g] = pltpu.async_copy(
            rows_v.at[g % 3], out_hbm.at[pl.ds(base + g * _CHUNK, _CHUNK)],
            sem_o)
        if g + 3 < n_chunks:
            odesc[g].wait()
            rdesc[g + 3] = fire_rows(g + 3)
    for g in range(max(0, n_chunks - 3), n_chunks):
        odesc[g].wait()


def kernel(memory, node_ids, values):
    del memory
    out, _ = _sc_scatter_gather(node_ids, values)
    return out
